# baseline (speedup 1.0000x reference)
import jax
import jax.numpy as jnp
from jax.experimental import pallas as pl


B, N, T, F_IN, F_OUT = 64, 325, 12, 64, 64

_G = 2
_BB = 16
_GW = _G * F_OUT


def _gcn_body(x_ref, adjt_ref, wbdt_ref, bc_ref, o_ref):
    for i in range(_BB):
        ht = jnp.dot(x_ref[i], adjt_ref[...], preferred_element_type=jnp.float32)
        s = jnp.dot(wbdt_ref[...], ht, preferred_element_type=jnp.float32)
        o_ref[i] = jnp.maximum(s + bc_ref[...], 0.0)


@jax.jit
def kernel(x, adj, W, b):
    xt = jnp.transpose(x, (0, 2, 3, 1)).reshape(B, T * F_IN, N)
    adjt = adj.T
    wbdt = jnp.kron(jnp.eye(_G, dtype=W.dtype), W).T
    bc = jnp.tile(b, _G).reshape(_G * F_OUT, 1)
    out = pl.pallas_call(
        _gcn_body,
        grid=(B // _BB, T // _G),
        in_specs=[
            pl.BlockSpec((_BB, _GW, N), lambda i, j: (i, j, 0)),
            pl.BlockSpec((N, N), lambda i, j: (0, 0)),
            pl.BlockSpec((_GW, _GW), lambda i, j: (0, 0)),
            pl.BlockSpec((_GW, 1), lambda i, j: (0, 0)),
        ],
        out_specs=pl.BlockSpec((_BB, _GW, N), lambda i, j: (i, j, 0)),
        out_shape=jax.ShapeDtypeStruct((B, T * F_OUT, N), jnp.float32),
    )(xt, adjt, wbdt, bc)
    return jnp.transpose(out.reshape(B, T, F_OUT, N), (0, 3, 1, 2))


# speedup vs baseline: 1.7778x; 1.7778x over previous
import jax
import jax.numpy as jnp
from jax.experimental import pallas as pl


B, N, T, F_IN, F_OUT = 64, 325, 12, 64, 64

_G = 2
_BB = 8


def _gcn_body(x_ref, adjt_ref, wbdt_ref, bc_ref, o_ref):
    gw = _G * F_OUT
    for i in range(_BB):
        ht = jnp.dot(x_ref[i], adjt_ref[...], preferred_element_type=jnp.float32)
        for j in range(T // _G):
            s = jnp.dot(wbdt_ref[...], ht[j * gw:(j + 1) * gw, :],
                        preferred_element_type=jnp.float32)
            o_ref[i, j * gw:(j + 1) * gw, :] = jnp.maximum(s + bc_ref[...], 0.0)


@jax.jit
def kernel(x, adj, W, b):
    xt = jnp.transpose(x, (0, 2, 3, 1)).reshape(B, T * F_IN, N)
    adjt = adj.T
    wbdt = jnp.kron(jnp.eye(_G, dtype=W.dtype), W).T
    bc = jnp.tile(b, _G).reshape(_G * F_OUT, 1)
    out = pl.pallas_call(
        _gcn_body,
        grid=(B // _BB,),
        in_specs=[
            pl.BlockSpec((_BB, T * F_IN, N), lambda i: (i, 0, 0)),
            pl.BlockSpec((N, N), lambda i: (0, 0)),
            pl.BlockSpec((_G * F_OUT, _G * F_IN), lambda i: (0, 0)),
            pl.BlockSpec((_G * F_OUT, 1), lambda i: (0, 0)),
        ],
        out_specs=pl.BlockSpec((_BB, T * F_OUT, N), lambda i: (i, 0, 0)),
        out_shape=jax.ShapeDtypeStruct((B, T * F_OUT, N), jnp.float32),
    )(xt, adjt, wbdt, bc)
    return jnp.transpose(out.reshape(B, T, F_OUT, N), (0, 3, 1, 2))
